# baseline (device time: 13139 ns/iter reference)
import os

import jax
import jax.numpy as jnp
from jax import lax
from jax.experimental import pallas as pl
from jax.experimental.pallas import tpu as pltpu

try:
    VARIANT = (
        open(os.path.join(os.path.dirname(__file__), "variant.txt")).read().strip()
        or "full"
    )
except OSError:
    VARIANT = "full"

N_DEV = 4
CHUNKS = (96, 96, 64)
HOP_ORDER = (2, 1, 3)


def kernel(x, Wg, Wu, Wd):
    m, _ = x.shape
    d = Wd.shape[1]
    n_chunk = len(CHUNKS)
    offs = [sum(CHUNKS[:c]) for c in range(n_chunk)]

    def body(x_ref, wg_ref, wu_ref, wd_ref, out_ref, *scratch):
        send_refs = scratch[:n_chunk]
        comm_refs = scratch[n_chunk:2 * n_chunk]
        send_sems, recv_sems = scratch[2 * n_chunk:]
        my_pos = lax.axis_index("i")

        barrier_sem = pltpu.get_barrier_semaphore()
        for h in range(1, N_DEV):
            pl.semaphore_signal(
                barrier_sem, inc=1,
                device_id=(lax.rem(my_pos + h, N_DEV),),
                device_id_type=pl.DeviceIdType.MESH,
            )

        xb = x_ref[...].astype(jnp.bfloat16)
        wgb = wg_ref[...].astype(jnp.bfloat16)
        wub = wu_ref[...].astype(jnp.bfloat16)
        wdb = wd_ref[...].astype(jnp.bfloat16)

        partials = []
        rdmas = []
        for c, sz in enumerate(CHUNKS):
            xc = xb[offs[c]:offs[c] + sz, :]
            gate = jnp.dot(xc, wgb, preferred_element_type=jnp.float32)
            up = jnp.dot(xc, wub, preferred_element_type=jnp.float32)
            hidden = (gate * (up * jax.nn.sigmoid(up))).astype(jnp.bfloat16)
            p = jnp.dot(hidden, wdb, preferred_element_type=jnp.float32)
            partials.append(p)
            send_refs[c][...] = p.astype(jnp.bfloat16)
            if c == 0:
                pl.semaphore_wait(barrier_sem, N_DEV - 1)
            if VARIANT == "barrier":
                continue
            for h in HOP_ORDER:
                rdma = pltpu.make_async_remote_copy(
                    src_ref=send_refs[c],
                    dst_ref=comm_refs[c].at[h - 1],
                    send_sem=send_sems.at[h - 1, c],
                    recv_sem=recv_sems.at[h - 1, c],
                    device_id=(lax.rem(my_pos + h, N_DEV),),
                    device_id_type=pl.DeviceIdType.MESH,
                )
                rdma.start()
                rdmas.append(rdma)

        for c, sz in enumerate(CHUNKS):
            if VARIANT != "barrier":
                for i in range(len(HOP_ORDER)):
                    rdmas[c * len(HOP_ORDER) + i].wait_recv()
            if VARIANT in ("nosum", "barrier"):
                out_ref[offs[c]:offs[c] + sz, :] = partials[c]
            else:
                out_ref[offs[c]:offs[c] + sz, :] = (
                    partials[c]
                    + comm_refs[c][0].astype(jnp.float32)
                    + comm_refs[c][1].astype(jnp.float32)
                    + comm_refs[c][2].astype(jnp.float32))

        for rdma in rdmas:
            rdma.wait_send()

    return pl.pallas_call(
        body,
        out_shape=jax.ShapeDtypeStruct((m, d), jnp.float32),
        in_specs=[pl.BlockSpec(memory_space=pltpu.VMEM)] * 4,
        out_specs=pl.BlockSpec(memory_space=pltpu.VMEM),
        scratch_shapes=(
            [pltpu.VMEM((sz, d), jnp.bfloat16) for sz in CHUNKS]
            + [pltpu.VMEM((N_DEV - 1, sz, d), jnp.bfloat16) for sz in CHUNKS]
            + [
                pltpu.SemaphoreType.DMA((N_DEV - 1, n_chunk)),
                pltpu.SemaphoreType.DMA((N_DEV - 1, n_chunk)),
            ]
        ),
        compiler_params=pltpu.CompilerParams(collective_id=0),
    )(x, Wg, Wu, Wd)


# device time: 8827 ns/iter; 1.4885x vs baseline; 1.4885x over previous
import os

import jax
import jax.numpy as jnp
from jax import lax
from jax.experimental import pallas as pl
from jax.experimental.pallas import tpu as pltpu

try:
    VARIANT = (
        open(os.path.join(os.path.dirname(__file__), "variant.txt")).read().strip()
        or "full"
    )
except OSError:
    VARIANT = "full"

N_DEV = 4
CHUNKS = (128, 128)
HOP_ORDER = (2, 1, 3)


def kernel(x, Wg, Wu, Wd):
    m, k = x.shape
    hdim = Wg.shape[1]
    d = Wd.shape[1]
    n_chunk = len(CHUNKS)
    offs = [sum(CHUNKS[:c]) for c in range(n_chunk)]

    def body(x_hbm, wg_hbm, wu_hbm, wd_hbm, out_ref, *scratch):
        xv, wgv, wuv, wdv = scratch[:4]
        in_sems = scratch[4]
        send_refs = scratch[5:5 + n_chunk]
        comm_refs = scratch[5 + n_chunk:5 + 2 * n_chunk]
        send_sems, recv_sems = scratch[5 + 2 * n_chunk:]

        my_pos = lax.axis_index("i")

        barrier_sem = pltpu.get_barrier_semaphore()
        for h in range(1, N_DEV):
            pl.semaphore_signal(
                barrier_sem, inc=1,
                device_id=(lax.rem(my_pos + h, N_DEV),),
                device_id_type=pl.DeviceIdType.MESH,
            )

        cps = [
            pltpu.make_async_copy(src, dst, in_sems.at[i])
            for i, (src, dst) in enumerate(
                [(x_hbm, xv), (wg_hbm, wgv), (wu_hbm, wuv), (wd_hbm, wdv)]
            )
        ]
        for cp in cps:
            cp.start()

        cps[0].wait()
        xb = xv[...].astype(jnp.bfloat16)
        cps[1].wait()
        gate = jnp.dot(xb, wgv[...].astype(jnp.bfloat16),
                       preferred_element_type=jnp.float32)
        cps[2].wait()
        up = jnp.dot(xb, wuv[...].astype(jnp.bfloat16),
                     preferred_element_type=jnp.float32)
        hidden = (gate * (up * jax.nn.sigmoid(up))).astype(jnp.bfloat16)
        cps[3].wait()
        wdb = wdv[...].astype(jnp.bfloat16)

        partials = []
        rdmas = []
        for c, sz in enumerate(CHUNKS):
            p = jnp.dot(hidden[offs[c]:offs[c] + sz, :], wdb,
                        preferred_element_type=jnp.float32)
            partials.append(p)
            send_refs[c][...] = p.astype(jnp.bfloat16)
            if c == 0:
                pl.semaphore_wait(barrier_sem, N_DEV - 1)
            if VARIANT == "barrier":
                continue
            for h in HOP_ORDER:
                rdma = pltpu.make_async_remote_copy(
                    src_ref=send_refs[c],
                    dst_ref=comm_refs[c].at[h - 1],
                    send_sem=send_sems.at[h - 1, c],
                    recv_sem=recv_sems.at[h - 1, c],
                    device_id=(lax.rem(my_pos + h, N_DEV),),
                    device_id_type=pl.DeviceIdType.MESH,
                )
                rdma.start()
                rdmas.append(rdma)

        for c, sz in enumerate(CHUNKS):
            if VARIANT != "barrier":
                for i in range(len(HOP_ORDER)):
                    rdmas[c * len(HOP_ORDER) + i].wait_recv()
            if VARIANT in ("nosum", "barrier"):
                out_ref[offs[c]:offs[c] + sz, :] = partials[c].astype(jnp.bfloat16)
            else:
                out_ref[offs[c]:offs[c] + sz, :] = (
                    partials[c]
                    + comm_refs[c][0].astype(jnp.float32)
                    + comm_refs[c][1].astype(jnp.float32)
                    + comm_refs[c][2].astype(jnp.float32)
                ).astype(jnp.bfloat16)

        for rdma in rdmas:
            rdma.wait_send()

    return pl.pallas_call(
        body,
        out_shape=jax.ShapeDtypeStruct((m, d), jnp.bfloat16),
        in_specs=[pl.BlockSpec(memory_space=pl.ANY)] * 4,
        out_specs=pl.BlockSpec(memory_space=pltpu.VMEM),
        scratch_shapes=(
            [
                pltpu.VMEM((m, k), jnp.float32),
                pltpu.VMEM((k, hdim), jnp.float32),
                pltpu.VMEM((k, hdim), jnp.float32),
                pltpu.VMEM((hdim, d), jnp.float32),
                pltpu.SemaphoreType.DMA((4,)),
            ]
            + [pltpu.VMEM((sz, d), jnp.bfloat16) for sz in CHUNKS]
            + [pltpu.VMEM((N_DEV - 1, sz, d), jnp.bfloat16) for sz in CHUNKS]
            + [
                pltpu.SemaphoreType.DMA((N_DEV - 1, n_chunk)),
                pltpu.SemaphoreType.DMA((N_DEV - 1, n_chunk)),
            ]
        ),
        compiler_params=pltpu.CompilerParams(collective_id=0),
    )(x, Wg, Wu, Wd)
